# baseline (speedup 1.0000x reference)
import jax
import jax.numpy as jnp
from jax.experimental import pallas as pl
from jax.experimental.pallas import tpu as pltpu

_TT = 16


def _base_kernel(eids_ref, regions_ref, b_ref, slot_ref, regemb_ref,
                 eidemb_ref, base_ref):
    b_idx = pl.program_id(0)
    n = slot_ref.shape[0]
    regions = regions_ref[0, :, :]
    nregions = regemb_ref.shape[0]
    oh = (regions == jax.lax.broadcasted_iota(
        jnp.int32, (n, nregions), 1)).astype(jnp.float32)
    reg = jnp.dot(oh, regemb_ref[...], preferred_element_type=jnp.float32)

    e = eids_ref[b_idx]
    neids = eidemb_ref.shape[0]
    ohe = (jax.lax.broadcasted_iota(jnp.int32, (8, neids), 1) == e
           ).astype(jnp.float32)
    ev = jnp.dot(ohe, eidemb_ref[...], preferred_element_type=jnp.float32)

    base_ref[0] = slot_ref[...] + reg + ev[0:1, :] + b_ref[...]


def _bcast_kernel(spikes_ref, idx_ref, w_ref, base_ref, out_ref):
    tt, nd = out_ref.shape[1], out_ref.shape[2]
    n = spikes_ref.shape[2]
    d = nd // n
    cw = 128 * d
    sp = spikes_ref[0]
    idx = jnp.broadcast_to(idx_ref[...], (tt, cw))
    for c in range(n // 128):
        chunk = sp[:, c * 128:(c + 1) * 128]
        s2 = jnp.take_along_axis(chunk, idx, axis=1)
        sl = pl.ds(c * cw, cw)
        out_ref[0, :, sl] = s2 * w_ref[0, sl] + base_ref[0, 0, sl]


@jax.jit
def kernel(spikes, neuron_regions, eids, w_spike, b_spike, neuron_slot,
           region_emb, eid_emb):
    B, T, N = spikes.shape
    D = neuron_slot.shape[1]
    ND = N * D
    regions3 = neuron_regions.astype(jnp.int32).reshape(B, N, 1)
    eids32 = eids.astype(jnp.int32)
    b2 = b_spike.reshape(1, D)

    base = pl.pallas_call(
        _base_kernel,
        grid=(B,),
        in_specs=[
            pl.BlockSpec(memory_space=pltpu.SMEM),
            pl.BlockSpec((1, N, 1), lambda b: (b, 0, 0)),
            pl.BlockSpec((1, D), lambda b: (0, 0)),
            pl.BlockSpec((N, D), lambda b: (0, 0)),
            pl.BlockSpec(region_emb.shape, lambda b: (0, 0)),
            pl.BlockSpec(eid_emb.shape, lambda b: (0, 0)),
        ],
        out_specs=pl.BlockSpec((1, N, D), lambda b: (b, 0, 0)),
        out_shape=jax.ShapeDtypeStruct((B, N, D), jnp.float32),
    )(eids32, regions3, b2, neuron_slot, region_emb, eid_emb)

    wflat = jnp.tile(w_spike[:, 0], N).reshape(1, ND)
    idxflat = (jnp.arange(128 * D, dtype=jnp.int32) // D).reshape(1, 128 * D)
    baseflat = base.reshape(B, 1, ND)

    out = pl.pallas_call(
        _bcast_kernel,
        grid=(B, T // _TT),
        in_specs=[
            pl.BlockSpec((1, _TT, N), lambda b, t: (b, t, 0)),
            pl.BlockSpec((1, 128 * D), lambda b, t: (0, 0)),
            pl.BlockSpec((1, ND), lambda b, t: (0, 0)),
            pl.BlockSpec((1, 1, ND), lambda b, t: (b, 0, 0)),
        ],
        out_specs=pl.BlockSpec((1, _TT, ND), lambda b, t: (b, t, 0)),
        out_shape=jax.ShapeDtypeStruct((B, T, ND), jnp.float32),
    )(spikes, idxflat, wflat, baseflat)
    return out.reshape(B, T, N, D)


# speedup vs baseline: 7.0225x; 7.0225x over previous
import jax
import jax.numpy as jnp
from jax.experimental import pallas as pl
from jax.experimental.pallas import tpu as pltpu

_TT = 16


def _fused_kernel(eids_ref, regions_ref, spikes_ref, wfull_ref, bcol_ref,
                  slott_ref, regembt_ref, eidembt_ref, out_ref, base_ref):
    b_idx = pl.program_id(0)
    t_idx = pl.program_id(1)
    d, n = base_ref.shape

    @pl.when(t_idx == 0)
    def _build_base():
        regions = regions_ref[0, :, :]
        nregions = regembt_ref.shape[1]
        oht = (jax.lax.broadcasted_iota(jnp.int32, (nregions, n), 0)
               == regions).astype(jnp.float32)
        regt = jnp.dot(regembt_ref[...], oht,
                       preferred_element_type=jnp.float32)

        e = eids_ref[b_idx]
        neids = eidembt_ref.shape[1]
        ohe = (jax.lax.broadcasted_iota(jnp.int32, (neids, 8), 0) == e
               ).astype(jnp.float32)
        evt = jnp.dot(eidembt_ref[...], ohe,
                      preferred_element_type=jnp.float32)

        base_ref[...] = (slott_ref[...] + regt
                         + evt[:, 0:1] + bcol_ref[...])

    sp = spikes_ref[0]
    tt = sp.shape[0]
    out_ref[0] = (sp[:, None, :] * wfull_ref[...][None, :, :]
                  + base_ref[...][None, :, :])


@jax.jit
def kernel(spikes, neuron_regions, eids, w_spike, b_spike, neuron_slot,
           region_emb, eid_emb):
    B, T, N = spikes.shape
    D = neuron_slot.shape[1]
    regions3 = neuron_regions.astype(jnp.int32).reshape(B, 1, N)
    eids32 = eids.astype(jnp.int32)
    wfull = jnp.broadcast_to(w_spike, (D, N))
    bcol = b_spike.reshape(D, 1)
    slott = neuron_slot[:N].T
    regembt = region_emb.T
    eidembt = eid_emb.T

    outt = pl.pallas_call(
        _fused_kernel,
        grid=(B, T // _TT),
        in_specs=[
            pl.BlockSpec(memory_space=pltpu.SMEM),
            pl.BlockSpec((1, 1, N), lambda b, t: (b, 0, 0)),
            pl.BlockSpec((1, _TT, N), lambda b, t: (b, t, 0)),
            pl.BlockSpec((D, N), lambda b, t: (0, 0)),
            pl.BlockSpec((D, 1), lambda b, t: (0, 0)),
            pl.BlockSpec((D, N), lambda b, t: (0, 0)),
            pl.BlockSpec((D, region_emb.shape[0]), lambda b, t: (0, 0)),
            pl.BlockSpec((D, eid_emb.shape[0]), lambda b, t: (0, 0)),
        ],
        out_specs=pl.BlockSpec((1, _TT, D, N), lambda b, t: (b, t, 0, 0)),
        out_shape=jax.ShapeDtypeStruct((B, T, D, N), jnp.float32),
        scratch_shapes=[pltpu.VMEM((D, N), jnp.float32)],
    )(eids32, regions3, spikes, wfull, bcol, slott, regembt, eidembt)
    return outt.transpose(0, 1, 3, 2)


# speedup vs baseline: 7.4654x; 1.0631x over previous
import jax
import jax.numpy as jnp
from jax.experimental import pallas as pl
from jax.experimental.pallas import tpu as pltpu

_TT = 32


def _fused_kernel(eids_ref, regions_ref, spikes_ref, wfull_ref, bcol_ref,
                  slott_ref, regembt_ref, eidembt_ref, out_ref, base_ref):
    b_idx = pl.program_id(0)
    t_idx = pl.program_id(1)
    d, n = base_ref.shape

    @pl.when(t_idx == 0)
    def _build_base():
        regions = regions_ref[0, :, :]
        nregions = regembt_ref.shape[1]
        oht = (jax.lax.broadcasted_iota(jnp.int32, (nregions, n), 0)
               == regions).astype(jnp.float32)
        regt = jnp.dot(regembt_ref[...], oht,
                       preferred_element_type=jnp.float32)

        e = eids_ref[b_idx]
        neids = eidembt_ref.shape[1]
        ohe = (jax.lax.broadcasted_iota(jnp.int32, (neids, 8), 0) == e
               ).astype(jnp.float32)
        evt = jnp.dot(eidembt_ref[...], ohe,
                      preferred_element_type=jnp.float32)

        base_ref[...] = (slott_ref[...] + regt
                         + evt[:, 0:1] + bcol_ref[...])

    sp = spikes_ref[0]
    tt = sp.shape[0]
    out_ref[0] = (sp[:, None, :] * wfull_ref[...][None, :, :]
                  + base_ref[...][None, :, :])


@jax.jit
def kernel(spikes, neuron_regions, eids, w_spike, b_spike, neuron_slot,
           region_emb, eid_emb):
    B, T, N = spikes.shape
    D = neuron_slot.shape[1]
    regions3 = neuron_regions.astype(jnp.int32).reshape(B, 1, N)
    eids32 = eids.astype(jnp.int32)
    wfull = jnp.broadcast_to(w_spike, (D, N))
    bcol = b_spike.reshape(D, 1)
    slott = neuron_slot[:N].T
    regembt = region_emb.T
    eidembt = eid_emb.T

    outt = pl.pallas_call(
        _fused_kernel,
        grid=(B, T // _TT),
        in_specs=[
            pl.BlockSpec(memory_space=pltpu.SMEM),
            pl.BlockSpec((1, 1, N), lambda b, t: (b, 0, 0)),
            pl.BlockSpec((1, _TT, N), lambda b, t: (b, t, 0)),
            pl.BlockSpec((D, N), lambda b, t: (0, 0)),
            pl.BlockSpec((D, 1), lambda b, t: (0, 0)),
            pl.BlockSpec((D, N), lambda b, t: (0, 0)),
            pl.BlockSpec((D, region_emb.shape[0]), lambda b, t: (0, 0)),
            pl.BlockSpec((D, eid_emb.shape[0]), lambda b, t: (0, 0)),
        ],
        out_specs=pl.BlockSpec((1, _TT, D, N), lambda b, t: (b, t, 0, 0)),
        out_shape=jax.ShapeDtypeStruct((B, T, D, N), jnp.float32),
        scratch_shapes=[pltpu.VMEM((D, N), jnp.float32)],
    )(eids32, regions3, spikes, wfull, bcol, slott, regembt, eidembt)
    return outt.transpose(0, 1, 3, 2)
